# Initial kernel scaffold; baseline (speedup 1.0000x reference)
import jax
import jax.numpy as jnp
from jax.experimental import pallas as pl


def kernel(tokens, x_bn, theta_W, alpha_raw):
    raise NotImplementedError("write your pallas kernel here")


# speedup vs baseline: 6.0094x; 6.0094x over previous
import jax
import jax.numpy as jnp
from jax.experimental import pallas as pl

_B, _N, _D, _L, _K = 4, 2048, 32, 128, 16
_NSTEPS = 4
_H = (0.2 - 0.0) / _NSTEPS


def _var_pde_kernel(x_bn_ref, tokens_ref, theta_ref, alpha_ref, out_ref):
    x = x_bn_ref[0]
    mu = jnp.mean(x, axis=1, keepdims=True)
    xc = x - mu
    var = jnp.sum(xc * xc, axis=1, keepdims=True) / (_L - 1)
    xn = xc / (jnp.sqrt(var) + 1e-6)
    corr = jax.lax.dot_general(
        xn, xn, (((1,), (1,)), ((), ())),
        preferred_element_type=jnp.float32) * (1.0 / (_L - 1))

    col = jax.lax.broadcasted_iota(jnp.int32, (_N, _N), 1)
    w = corr
    for _ in range(_K + 1):
        m = jnp.max(w, axis=1, keepdims=True)
        pos = jnp.min(jnp.where(w >= m, col, _N), axis=1, keepdims=True)
        sel = col == pos
        w = jnp.where(sel, -w - 10.0, w)
    a = jnp.where(w < -2.0, -(w + 10.0), 0.0)

    a = 0.5 * (a + a.T)
    row = jax.lax.broadcasted_iota(jnp.int32, (_N, _N), 0)
    a = a + jnp.where(row == col, 1.0, 0.0)
    deg = jnp.maximum(jnp.sum(a, axis=1, keepdims=True), 1e-6)
    dinv = jax.lax.rsqrt(deg)
    a = a * dinv * dinv.reshape(1, _N)

    alpha = alpha_ref[0, 0]
    th = theta_ref[:]

    def rhs(v):
        p = jnp.dot(a, v, preferred_element_type=jnp.float32)
        r = jax.lax.dot_general(
            p, th, (((1,), (1,)), ((), ())),
            preferred_element_type=jnp.float32)
        return alpha * (v - p) + r

    y = tokens_ref[0]
    for _ in range(_NSTEPS):
        k1 = rhs(y)
        k2 = rhs(y + (0.5 * _H) * k1)
        k3 = rhs(y + (0.5 * _H) * k2)
        k4 = rhs(y + _H * k3)
        y = y + (_H / 6.0) * (k1 + 2.0 * k2 + 2.0 * k3 + k4)
    out_ref[0] = jnp.maximum(y, 0.0)


@jax.jit
def kernel(tokens, x_bn, theta_W, alpha_raw):
    alpha = jnp.minimum(jax.nn.softplus(alpha_raw), 2.0).reshape(1, 1)
    return pl.pallas_call(
        _var_pde_kernel,
        grid=(_B,),
        in_specs=[
            pl.BlockSpec((1, _N, _L), lambda b: (b, 0, 0)),
            pl.BlockSpec((1, _N, _D), lambda b: (b, 0, 0)),
            pl.BlockSpec((_D, _D), lambda b: (0, 0)),
            pl.BlockSpec((1, 1), lambda b: (0, 0)),
        ],
        out_specs=pl.BlockSpec((1, _N, _D), lambda b: (b, 0, 0)),
        out_shape=jax.ShapeDtypeStruct((_B, _N, _D), jnp.float32),
    )(x_bn, tokens, theta_W, alpha)
